# Initial kernel scaffold; baseline (speedup 1.0000x reference)
import jax
import jax.numpy as jnp
from jax.experimental import pallas as pl


def kernel(padded_input, input_lengths, padded_target, W_enc, w_assign, W_out):
    raise NotImplementedError("write your pallas kernel here")


# speedup vs baseline: 33.4490x; 33.4490x over previous
import jax
import jax.numpy as jnp
from jax import lax
from jax.experimental import pallas as pl
from jax.experimental.pallas import tpu as pltpu

_B, _T, _D, _H, _TO, _V = 16, 2048, 512, 512, 256, 1000
_THR = 0.95
_INTERPRET = False


def _enc_body(len_ref, x_ref, wenc_ref, enc_ref):
    b = pl.program_id(0)
    L = len_ref[b]
    x = x_ref[0]
    enc = jnp.tanh(jnp.dot(x, wenc_ref[...], preferred_element_type=jnp.float32))
    rowmask = (lax.broadcasted_iota(jnp.int32, (_T, 1), 0) < L).astype(jnp.float32)
    enc_ref[0] = enc * rowmask


def _scan_body(aT_ref, sT_ref, nT_ref):
    def step(t, carry):
        s, n = carry
        a = aT_ref[pl.ds(t, 1), :]
        rec = s + a
        fire = rec > _THR
        s2 = jnp.where(fire, rec - 1.0, rec)
        n2 = n + fire.astype(jnp.int32)
        sT_ref[pl.ds(t, 1), :] = s2
        nT_ref[pl.ds(t, 1), :] = n2
        return (s2, n2)

    s0 = jnp.zeros((1, _B), jnp.float32)
    n0 = jnp.zeros((1, _B), jnp.int32)
    lax.fori_loop(0, _T, step, (s0, n0))


def _out_body(alpha_ref, sprev_ref, n_ref, enc_ref, wout_ref, pred_ref):
    alpha = alpha_ref[0]
    s_prev = sprev_ref[0]
    n = n_ref[0]
    rec = s_prev + alpha
    fire = rec > _THR
    seg = n - fire.astype(jnp.int32)
    dc = 1.0 - s_prev
    cur = jnp.where(fire, dc, alpha)
    rem = alpha - cur
    nfin = jnp.max(n)

    j2 = lax.broadcasted_iota(jnp.int32, (_TO, _T), 0)
    segB = jnp.broadcast_to(seg, (_TO, _T))
    M = jnp.where(segB == j2, jnp.broadcast_to(cur, (_TO, _T)), 0.0)
    M = M + jnp.where(segB + 1 == j2, jnp.broadcast_to(rem, (_TO, _T)), 0.0)
    M = jnp.where(j2 < nfin, M, 0.0)

    l = lax.dot_general(M, enc_ref[0], (((1,), (0,)), ((), ())),
                        precision=lax.Precision.HIGHEST,
                        preferred_element_type=jnp.float32)
    pred_ref[0] = jnp.dot(l, wout_ref[...], preferred_element_type=jnp.float32)


def kernel(padded_input, input_lengths, padded_target, W_enc, w_assign, W_out):
    lengths = input_lengths.astype(jnp.int32)

    enc = pl.pallas_call(
        _enc_body,
        grid=(_B,),
        in_specs=[
            pl.BlockSpec(memory_space=pltpu.SMEM),
            pl.BlockSpec((1, _T, _D), lambda b: (b, 0, 0)),
            pl.BlockSpec((_D, _H), lambda b: (0, 0)),
        ],
        out_specs=pl.BlockSpec((1, _T, _H), lambda b: (b, 0, 0)),
        out_shape=jax.ShapeDtypeStruct((_B, _T, _H), jnp.float32),
        interpret=_INTERPRET,
    )(lengths, padded_input, W_enc)

    t_idx = jnp.arange(_T)
    len_mask = (t_idx[None, :] < input_lengths[:, None]).astype(jnp.float32)
    alpha = jax.nn.sigmoid(jnp.einsum('bth,h->bt', enc, w_assign)) * len_mask
    _num = alpha.sum(-1)
    num = (padded_target > 0).astype(jnp.float32).sum(-1)
    alpha = alpha * (num / _num)[:, None]

    alphaT = jnp.swapaxes(alpha, 0, 1)
    sT, nT = pl.pallas_call(
        _scan_body,
        out_shape=[
            jax.ShapeDtypeStruct((_T, _B), jnp.float32),
            jax.ShapeDtypeStruct((_T, _B), jnp.int32),
        ],
        interpret=_INTERPRET,
    )(alphaT)

    s = jnp.swapaxes(sT, 0, 1)
    n = jnp.swapaxes(nT, 0, 1)
    s_prev = jnp.concatenate(
        [jnp.zeros((_B, 1), jnp.float32), s[:, :-1]], axis=1)

    alpha3 = alpha.reshape(_B, 1, _T)
    sprev3 = s_prev.reshape(_B, 1, _T)
    n3 = n.reshape(_B, 1, _T)
    wout_pad = jnp.pad(W_out, ((0, 0), (0, 24)))

    pred_pad = pl.pallas_call(
        _out_body,
        grid=(_B,),
        in_specs=[
            pl.BlockSpec((1, 1, _T), lambda b: (b, 0, 0)),
            pl.BlockSpec((1, 1, _T), lambda b: (b, 0, 0)),
            pl.BlockSpec((1, 1, _T), lambda b: (b, 0, 0)),
            pl.BlockSpec((1, _T, _H), lambda b: (b, 0, 0)),
            pl.BlockSpec((_H, 1024), lambda b: (0, 0)),
        ],
        out_specs=pl.BlockSpec((1, _TO, 1024), lambda b: (b, 0, 0)),
        out_shape=jax.ShapeDtypeStruct((_B, _TO, 1024), jnp.float32),
        interpret=_INTERPRET,
    )(alpha3, sprev3, n3, enc, wout_pad)

    pred = pred_pad[:, :, :_V]
    return pred, padded_target


# speedup vs baseline: 34.4319x; 1.0294x over previous
import functools

import jax
import jax.numpy as jnp
from jax import lax
from jax.experimental import pallas as pl
from jax.experimental.pallas import tpu as pltpu
from jax.experimental.pallas import tpu_sc as plsc

_B, _T, _D, _H, _TO, _V = 16, 2048, 512, 512, 256, 1000
_THR = 0.95
_INTERPRET = False


def _enc_body(len_ref, x_ref, wenc_ref, enc_ref):
    b = pl.program_id(0)
    L = len_ref[b]
    x = x_ref[0]
    enc = jnp.tanh(jnp.dot(x, wenc_ref[...], preferred_element_type=jnp.float32))
    rowmask = (lax.broadcasted_iota(jnp.int32, (_T, 1), 0) < L).astype(jnp.float32)
    enc_ref[0] = enc * rowmask


def _sc_scan_body(aT_hbm, s_hbm, n_hbm, a_v, s_v, n_v):
    wid = lax.axis_index("s") * 2 + lax.axis_index("c")

    @pl.when(wid == 0)
    def _():
        pltpu.sync_copy(aT_hbm, a_v)

        ones_f = jnp.full((_B,), 1.0, jnp.float32)
        zeros_f = jnp.zeros((_B,), jnp.float32)
        ones_i = jnp.full((_B,), 1, jnp.int32)
        zeros_i = jnp.zeros((_B,), jnp.int32)
        thr_v = jnp.full((_B,), _THR, jnp.float32)

        def step(t, carry):
            s, n = carry
            a = a_v[pl.ds(t * _B, _B)]
            rec = s + a
            fire = rec > thr_v
            s2 = rec - jnp.where(fire, ones_f, zeros_f)
            n2 = n + jnp.where(fire, ones_i, zeros_i)
            s_v[pl.ds(t * _B, _B)] = s2
            n_v[pl.ds(t * _B, _B)] = n2
            return (s2, n2)

        lax.fori_loop(0, _T, step,
                      (jnp.zeros((_B,), jnp.float32), jnp.zeros((_B,), jnp.int32)))
        pltpu.sync_copy(s_v, s_hbm)
        pltpu.sync_copy(n_v, n_hbm)


_sc_scan = functools.partial(
    pl.kernel,
    out_type=[
        jax.ShapeDtypeStruct((_T * _B,), jnp.float32),
        jax.ShapeDtypeStruct((_T * _B,), jnp.int32),
    ],
    mesh=plsc.VectorSubcoreMesh(core_axis_name="c", subcore_axis_name="s"),
    scratch_types=[
        pltpu.VMEM((_T * _B,), jnp.float32),
        pltpu.VMEM((_T * _B,), jnp.float32),
        pltpu.VMEM((_T * _B,), jnp.int32),
    ],
)(_sc_scan_body)


def _out_body(alpha_ref, sprev_ref, n_ref, enc_ref, wout_ref, pred_ref):
    alpha = alpha_ref[0]
    s_prev = sprev_ref[0]
    n = n_ref[0]
    rec = s_prev + alpha
    fire = rec > _THR
    seg = n - fire.astype(jnp.int32)
    dc = 1.0 - s_prev
    cur = jnp.where(fire, dc, alpha)
    rem = alpha - cur
    nfin = jnp.max(n)

    j2 = lax.broadcasted_iota(jnp.int32, (_TO, _T), 0)
    segB = jnp.broadcast_to(seg, (_TO, _T))
    M = jnp.where(segB == j2, jnp.broadcast_to(cur, (_TO, _T)), 0.0)
    M = M + jnp.where(segB + 1 == j2, jnp.broadcast_to(rem, (_TO, _T)), 0.0)
    M = jnp.where(j2 < nfin, M, 0.0)

    l = lax.dot_general(M, enc_ref[0], (((1,), (0,)), ((), ())),
                        precision=lax.Precision.HIGHEST,
                        preferred_element_type=jnp.float32)
    pred_ref[0] = jnp.dot(l, wout_ref[...], preferred_element_type=jnp.float32)


def kernel(padded_input, input_lengths, padded_target, W_enc, w_assign, W_out):
    lengths = input_lengths.astype(jnp.int32)

    enc = pl.pallas_call(
        _enc_body,
        grid=(_B,),
        in_specs=[
            pl.BlockSpec(memory_space=pltpu.SMEM),
            pl.BlockSpec((1, _T, _D), lambda b: (b, 0, 0)),
            pl.BlockSpec((_D, _H), lambda b: (0, 0)),
        ],
        out_specs=pl.BlockSpec((1, _T, _H), lambda b: (b, 0, 0)),
        out_shape=jax.ShapeDtypeStruct((_B, _T, _H), jnp.float32),
        interpret=_INTERPRET,
    )(lengths, padded_input, W_enc)

    t_idx = jnp.arange(_T)
    len_mask = (t_idx[None, :] < input_lengths[:, None]).astype(jnp.float32)
    alpha = jax.nn.sigmoid(jnp.einsum('bth,h->bt', enc, w_assign)) * len_mask
    _num = alpha.sum(-1)
    num = (padded_target > 0).astype(jnp.float32).sum(-1)
    alpha = alpha * (num / _num)[:, None]

    alphaT = jnp.swapaxes(alpha, 0, 1).reshape(_T * _B)
    sT, nT = _sc_scan(alphaT)

    s = jnp.swapaxes(sT.reshape(_T, _B), 0, 1)
    n = jnp.swapaxes(nT.reshape(_T, _B), 0, 1)
    s_prev = jnp.concatenate(
        [jnp.zeros((_B, 1), jnp.float32), s[:, :-1]], axis=1)

    alpha3 = alpha.reshape(_B, 1, _T)
    sprev3 = s_prev.reshape(_B, 1, _T)
    n3 = n.reshape(_B, 1, _T)
    wout_pad = jnp.pad(W_out, ((0, 0), (0, 24)))

    pred_pad = pl.pallas_call(
        _out_body,
        grid=(_B,),
        in_specs=[
            pl.BlockSpec((1, 1, _T), lambda b: (b, 0, 0)),
            pl.BlockSpec((1, 1, _T), lambda b: (b, 0, 0)),
            pl.BlockSpec((1, 1, _T), lambda b: (b, 0, 0)),
            pl.BlockSpec((1, _T, _H), lambda b: (b, 0, 0)),
            pl.BlockSpec((_H, 1024), lambda b: (0, 0)),
        ],
        out_specs=pl.BlockSpec((1, _TO, 1024), lambda b: (b, 0, 0)),
        out_shape=jax.ShapeDtypeStruct((_B, _TO, 1024), jnp.float32),
        interpret=_INTERPRET,
    )(alpha3, sprev3, n3, enc, wout_pad)

    pred = pred_pad[:, :, :_V]
    return pred, padded_target
